# Initial kernel scaffold; baseline (speedup 1.0000x reference)
import jax
import jax.numpy as jnp
from jax.experimental import pallas as pl


def kernel(x, edge_index, Wg, bg, W1, b1, W2, b2):
    raise NotImplementedError("write your pallas kernel here")


# speedup vs baseline: 10.7788x; 10.7788x over previous
import functools

import jax
import jax.numpy as jnp
from jax import lax
from jax.experimental import pallas as pl
from jax.experimental.pallas import tpu as pltpu
from jax.experimental.pallas import tpu_sc as plsc

N = 10000
E = 160000
D_IN = 128
D_H = 32
EC = 8000
KBLK = 16000


def _mm1_body(x_ref, wg_ref, o_ref):
    o_ref[...] = jnp.dot(x_ref[...], wg_ref[...],
                         preferred_element_type=jnp.float32)


def _k1(x, Wg):
    return pl.pallas_call(
        _mm1_body,
        out_shape=jax.ShapeDtypeStruct((N, D_H), jnp.float32),
    )(x, Wg)


_MESH = plsc.VectorSubcoreMesh(core_axis_name="c", subcore_axis_name="s")


@functools.partial(
    pl.kernel,
    mesh=_MESH,
    compiler_params=pltpu.CompilerParams(needs_layout_passes=False),
    out_type=jax.ShapeDtypeStruct((D_H, N), jnp.float32),
    scratch_types=[
        pltpu.VMEM((EC,), jnp.int32),
        pltpu.VMEM((EC,), jnp.int32),
        pltpu.VMEM((N,), jnp.float32),
        pltpu.VMEM((N,), jnp.float32),
        pltpu.VMEM((N,), jnp.float32),
        pltpu.VMEM((D_H,), jnp.float32),
    ],
)
def _sc_gcn(src_hbm, dst_hbm, xwT_hbm, bg_hbm, out_hbm,
            src_v, dst_v, dinv_v, xw_v, acc_v, bg_v):
    wid = lax.axis_index("s") * 2 + lax.axis_index("c")

    pltpu.sync_copy(xwT_hbm.at[wid], xw_v)
    pltpu.sync_copy(bg_hbm, bg_v)

    one16 = jnp.full((16,), 1.0, jnp.float32)
    zero16 = jnp.zeros((16,), jnp.float32)

    def _init(i, c):
        idx = pl.ds(i * 16, 16)
        dinv_v[idx] = one16
        acc_v[idx] = zero16
        return c
    lax.fori_loop(0, N // 16, _init, 0)

    def _deg_chunk(ci, c):
        pltpu.sync_copy(dst_hbm.at[pl.ds(ci * EC, EC)], dst_v)

        def _body(i, cc):
            d16 = dst_v[pl.ds(i * 16, 16)]
            plsc.addupdate_scatter(dinv_v, [d16], one16)
            return cc
        lax.fori_loop(0, EC // 16, _body, 0)
        return c
    lax.fori_loop(0, E // EC, _deg_chunk, 0)

    def _rsqrt(i, c):
        idx = pl.ds(i * 16, 16)
        v = dinv_v[idx]
        yi = jnp.int32(0x5F3759DF) - (plsc.bitcast(v, jnp.int32) >> 1)
        y = plsc.bitcast(yi, jnp.float32)
        h = v * 0.5
        y = y * (1.5 - h * y * y)
        y = y * (1.5 - h * y * y)
        y = y * (1.5 - h * y * y)
        dinv_v[idx] = y
        return c
    lax.fori_loop(0, N // 16, _rsqrt, 0)

    def _msg_chunk(ci, c):
        pltpu.sync_copy(src_hbm.at[pl.ds(ci * EC, EC)], src_v)
        pltpu.sync_copy(dst_hbm.at[pl.ds(ci * EC, EC)], dst_v)

        def _body(i, cc):
            s16 = src_v[pl.ds(i * 16, 16)]
            d16 = dst_v[pl.ds(i * 16, 16)]
            a = plsc.load_gather(dinv_v, [s16])
            b = plsc.load_gather(dinv_v, [d16])
            v = plsc.load_gather(xw_v, [s16])
            plsc.addupdate_scatter(acc_v, [d16], a * b * v)
            return cc
        lax.fori_loop(0, EC // 16, _body, 0)
        return c
    lax.fori_loop(0, E // EC, _msg_chunk, 0)

    widv = jnp.full((16,), 0, jnp.int32) + wid
    bgd = plsc.load_gather(bg_v, [widv])

    def _fin(i, c):
        idx = pl.ds(i * 16, 16)
        di = dinv_v[idx]
        h = acc_v[idx] + di * di * xw_v[idx] + bgd
        acc_v[idx] = jnp.maximum(h, 0.0)
        return c
    lax.fori_loop(0, N // 16, _fin, 0)

    pltpu.sync_copy(acc_v, out_hbm.at[wid])


def _fc1_body(h_ref, w_ref, o_ref):
    i = pl.program_id(0)

    @pl.when(i == 0)
    def _():
        o_ref[...] = jnp.zeros_like(o_ref)

    o_ref[...] += jnp.dot(h_ref[...], w_ref[...],
                          preferred_element_type=jnp.float32)


def _k3(h2, W1):
    nblk = (N * D_H) // KBLK
    return pl.pallas_call(
        _fc1_body,
        grid=(nblk,),
        in_specs=[
            pl.BlockSpec((1, KBLK), lambda i: (0, i)),
            pl.BlockSpec((KBLK, 128), lambda i: (i, 0)),
        ],
        out_specs=pl.BlockSpec((1, 128), lambda i: (0, 0)),
        out_shape=jax.ShapeDtypeStruct((1, 128), jnp.float32),
    )(h2, W1)


def _fc2_body(h_ref, b1_ref, w2_ref, b2_ref, o_ref):
    h3 = jnp.maximum(h_ref[...] + b1_ref[...], 0.0)
    o_ref[...] = jnp.maximum(
        jnp.dot(h3, w2_ref[...], preferred_element_type=jnp.float32)
        + b2_ref[...], 0.0)


def _k4(h3pre, b1, W2, b2):
    return pl.pallas_call(
        _fc2_body,
        out_shape=jax.ShapeDtypeStruct((1, N), jnp.float32),
    )(h3pre, b1.reshape(1, 128), W2, b2.reshape(1, N))


def kernel(x, edge_index, Wg, bg, W1, b1, W2, b2):
    src = edge_index[0]
    dst = edge_index[1]
    xw = _k1(x, Wg)
    h1T = _sc_gcn(src, dst, xw.T, bg)
    h2 = h1T.T.reshape(1, N * D_H)
    h3pre = _k3(h2, W1)
    return _k4(h3pre, b1, W2, b2)


# speedup vs baseline: 13.8867x; 1.2883x over previous
import functools

import jax
import jax.numpy as jnp
from jax import lax
from jax.experimental import pallas as pl
from jax.experimental.pallas import tpu as pltpu
from jax.experimental.pallas import tpu_sc as plsc

N = 10000
E = 160000
D_IN = 128
D_H = 32
EC = 8000
KBLK = 16000


def _mm1_body(x_ref, wg_ref, o_ref):
    o_ref[...] = jnp.dot(x_ref[...], wg_ref[...],
                         preferred_element_type=jnp.float32)


def _k1(x, Wg):
    return pl.pallas_call(
        _mm1_body,
        out_shape=jax.ShapeDtypeStruct((N, D_H), jnp.float32),
    )(x, Wg)


_MESH = plsc.VectorSubcoreMesh(core_axis_name="c", subcore_axis_name="s")

_NCH = E // EC
_U = 4


@functools.partial(
    pl.kernel,
    mesh=_MESH,
    compiler_params=pltpu.CompilerParams(needs_layout_passes=False),
    out_type=jax.ShapeDtypeStruct((D_H, N), jnp.float32),
    scratch_types=[
        pltpu.VMEM((EC,), jnp.int32),
        pltpu.VMEM((EC,), jnp.int32),
        pltpu.VMEM((EC,), jnp.int32),
        pltpu.VMEM((EC,), jnp.int32),
        pltpu.VMEM((N,), jnp.float32),
        pltpu.VMEM((N,), jnp.float32),
        pltpu.VMEM((N,), jnp.float32),
        pltpu.VMEM((D_H,), jnp.float32),
        pltpu.SemaphoreType.DMA,
        pltpu.SemaphoreType.DMA,
        pltpu.SemaphoreType.DMA,
        pltpu.SemaphoreType.DMA,
    ],
)
def _sc_gcn(src_hbm, dst_hbm, xwT_hbm, bg_hbm, out_hbm,
            src_v0, src_v1, dst_v0, dst_v1, dinv_v, g_v, acc_v, bg_v,
            sem_s0, sem_s1, sem_d0, sem_d1):
    wid = lax.axis_index("s") * 2 + lax.axis_index("c")
    src_b = (src_v0, src_v1)
    dst_b = (dst_v0, dst_v1)
    sem_s = (sem_s0, sem_s1)
    sem_d = (sem_d0, sem_d1)

    pltpu.sync_copy(xwT_hbm.at[wid], g_v)
    pltpu.sync_copy(bg_hbm, bg_v)

    one16 = jnp.full((16,), 1.0, jnp.float32)
    zero16 = jnp.zeros((16,), jnp.float32)

    def _init(i, c):
        idx = pl.ds(i * 16, 16)
        dinv_v[idx] = one16
        acc_v[idx] = zero16
        return c
    lax.fori_loop(0, N // 16, _init, 0)

    h = pltpu.async_copy(dst_hbm.at[pl.ds(0, EC)], dst_b[0], sem_d[0])
    for ci in range(_NCH):
        buf = dst_b[ci % 2]
        hn = None
        if ci + 1 < _NCH:
            nb = (ci + 1) % 2
            hn = pltpu.async_copy(dst_hbm.at[pl.ds((ci + 1) * EC, EC)],
                                  dst_b[nb], sem_d[nb])
        h.wait()

        def _body(i, cc, buf=buf):
            for u in range(_U):
                d16 = buf[pl.ds((i * _U + u) * 16, 16)]
                plsc.addupdate_scatter(dinv_v, [d16], one16)
            return cc
        lax.fori_loop(0, EC // 16 // _U, _body, 0)
        h = hn

    def _rsqrt(i, c):
        idx = pl.ds(i * 16, 16)
        v = dinv_v[idx]
        yi = jnp.int32(0x5F3759DF) - (plsc.bitcast(v, jnp.int32) >> 1)
        y = plsc.bitcast(yi, jnp.float32)
        hf = v * 0.5
        y = y * (1.5 - hf * y * y)
        y = y * (1.5 - hf * y * y)
        y = y * (1.5 - hf * y * y)
        dinv_v[idx] = y
        g_v[idx] = y * g_v[idx]
        return c
    lax.fori_loop(0, N // 16, _rsqrt, 0)

    hs = pltpu.async_copy(src_hbm.at[pl.ds(0, EC)], src_b[0], sem_s[0])
    hd = pltpu.async_copy(dst_hbm.at[pl.ds(0, EC)], dst_b[0], sem_d[0])
    for ci in range(_NCH):
        sbuf = src_b[ci % 2]
        dbuf = dst_b[ci % 2]
        hsn = hdn = None
        if ci + 1 < _NCH:
            nb = (ci + 1) % 2
            off = (ci + 1) * EC
            hsn = pltpu.async_copy(src_hbm.at[pl.ds(off, EC)],
                                   src_b[nb], sem_s[nb])
            hdn = pltpu.async_copy(dst_hbm.at[pl.ds(off, EC)],
                                   dst_b[nb], sem_d[nb])
        hs.wait()
        hd.wait()

        def _body(i, cc, sbuf=sbuf, dbuf=dbuf):
            for u in range(_U):
                idx = pl.ds((i * _U + u) * 16, 16)
                s16 = sbuf[idx]
                d16 = dbuf[idx]
                v = plsc.load_gather(g_v, [s16])
                plsc.addupdate_scatter(acc_v, [d16], v)
            return cc
        lax.fori_loop(0, EC // 16 // _U, _body, 0)
        hs, hd = hsn, hdn

    widv = jnp.full((16,), 0, jnp.int32) + wid
    bgd = plsc.load_gather(bg_v, [widv])

    def _fin(i, c):
        idx = pl.ds(i * 16, 16)
        hv = dinv_v[idx] * (acc_v[idx] + g_v[idx]) + bgd
        acc_v[idx] = jnp.maximum(hv, 0.0)
        return c
    lax.fori_loop(0, N // 16, _fin, 0)

    pltpu.sync_copy(acc_v, out_hbm.at[wid])


def _fc1_body(h_ref, w_ref, o_ref):
    i = pl.program_id(0)

    @pl.when(i == 0)
    def _():
        o_ref[...] = jnp.zeros_like(o_ref)

    o_ref[...] += jnp.dot(h_ref[...], w_ref[...],
                          preferred_element_type=jnp.float32)


def _k3(h2, W1):
    nblk = (N * D_H) // KBLK
    return pl.pallas_call(
        _fc1_body,
        grid=(nblk,),
        in_specs=[
            pl.BlockSpec((1, KBLK), lambda i: (0, i)),
            pl.BlockSpec((KBLK, 128), lambda i: (i, 0)),
        ],
        out_specs=pl.BlockSpec((1, 128), lambda i: (0, 0)),
        out_shape=jax.ShapeDtypeStruct((1, 128), jnp.float32),
    )(h2, W1)


def _fc2_body(h_ref, b1_ref, w2_ref, b2_ref, o_ref):
    h3 = jnp.maximum(h_ref[...] + b1_ref[...], 0.0)
    o_ref[...] = jnp.maximum(
        jnp.dot(h3, w2_ref[...], preferred_element_type=jnp.float32)
        + b2_ref[...], 0.0)


def _k4(h3pre, b1, W2, b2):
    return pl.pallas_call(
        _fc2_body,
        out_shape=jax.ShapeDtypeStruct((1, N), jnp.float32),
    )(h3pre, b1.reshape(1, 128), W2, b2.reshape(1, N))


def kernel(x, edge_index, Wg, bg, W1, b1, W2, b2):
    src = edge_index[0]
    dst = edge_index[1]
    xw = _k1(x, Wg)
    h1T = _sc_gcn(src, dst, xw.T, bg)
    h2 = h1T.T.reshape(1, N * D_H)
    h3pre = _k3(h2, W1)
    return _k4(h3pre, b1, W2, b2)


# speedup vs baseline: 21.8590x; 1.5741x over previous
import functools

import jax
import jax.numpy as jnp
from jax import lax
from jax.experimental import pallas as pl
from jax.experimental.pallas import tpu as pltpu
from jax.experimental.pallas import tpu_sc as plsc

N = 10000
E = 160000
D_IN = 128
D_H = 32
EC = 8000
KBLK = 16000


def _mm1_body(x_ref, wg_ref, o_ref):
    o_ref[...] = jnp.dot(x_ref[...], wg_ref[...],
                         preferred_element_type=jnp.float32)


def _k1(x, Wg):
    return pl.pallas_call(
        _mm1_body,
        out_shape=jax.ShapeDtypeStruct((N, D_H), jnp.float32),
    )(x, Wg)


_MESH = plsc.VectorSubcoreMesh(core_axis_name="c", subcore_axis_name="s")

_NCH = E // EC
_U = 8


@functools.partial(
    pl.kernel,
    mesh=_MESH,
    compiler_params=pltpu.CompilerParams(needs_layout_passes=False),
    out_type=jax.ShapeDtypeStruct((D_H, N), jnp.float32),
    scratch_types=[
        pltpu.VMEM((EC,), jnp.int32),
        pltpu.VMEM((EC,), jnp.int32),
        pltpu.VMEM((EC,), jnp.int32),
        pltpu.VMEM((EC,), jnp.int32),
        pltpu.VMEM((N,), jnp.float32),
        pltpu.VMEM((N,), jnp.float32),
        pltpu.VMEM((N,), jnp.float32),
        pltpu.VMEM((D_H,), jnp.float32),
        pltpu.SemaphoreType.DMA,
        pltpu.SemaphoreType.DMA,
        pltpu.SemaphoreType.DMA,
        pltpu.SemaphoreType.DMA,
    ],
)
def _sc_gcn(src_hbm, dst_hbm, xwT_hbm, bg_hbm, out_hbm,
            src_v0, src_v1, dst_v0, dst_v1, dinv_v, g_v, acc_v, bg_v,
            sem_s0, sem_s1, sem_d0, sem_d1):
    wid = lax.axis_index("s") * 2 + lax.axis_index("c")
    src_b = (src_v0, src_v1)
    dst_b = (dst_v0, dst_v1)
    sem_s = (sem_s0, sem_s1)
    sem_d = (sem_d0, sem_d1)

    pltpu.sync_copy(xwT_hbm.at[wid], g_v)
    pltpu.sync_copy(bg_hbm, bg_v)

    one16 = jnp.full((16,), 1.0, jnp.float32)
    zero16 = jnp.zeros((16,), jnp.float32)

    def _init(i, c):
        idx = pl.ds(i * 16, 16)
        dinv_v[idx] = one16
        acc_v[idx] = zero16
        return c
    lax.fori_loop(0, N // 16, _init, 0)

    h = pltpu.async_copy(dst_hbm.at[pl.ds(0, EC)], dst_b[0], sem_d[0])
    for ci in range(_NCH):
        buf = dst_b[ci % 2]
        hn = None
        if ci + 1 < _NCH:
            nb = (ci + 1) % 2
            hn = pltpu.async_copy(dst_hbm.at[pl.ds((ci + 1) * EC, EC)],
                                  dst_b[nb], sem_d[nb])
        h.wait()

        def _body(i, cc, buf=buf):
            d16s = [buf[pl.ds((i * _U + u) * 16, 16)] for u in range(_U)]
            for d16 in d16s:
                plsc.addupdate_scatter(dinv_v, [d16], one16)
            return cc
        lax.fori_loop(0, EC // 16 // _U, _body, 0)
        h = hn

    def _rsqrt(i, c):
        idx = pl.ds(i * 16, 16)
        v = dinv_v[idx]
        yi = jnp.int32(0x5F3759DF) - (plsc.bitcast(v, jnp.int32) >> 1)
        y = plsc.bitcast(yi, jnp.float32)
        hf = v * 0.5
        y = y * (1.5 - hf * y * y)
        y = y * (1.5 - hf * y * y)
        y = y * (1.5 - hf * y * y)
        dinv_v[idx] = y
        g_v[idx] = y * g_v[idx]
        return c
    lax.fori_loop(0, N // 16, _rsqrt, 0)

    hs = pltpu.async_copy(src_hbm.at[pl.ds(0, EC)], src_b[0], sem_s[0])
    hd = pltpu.async_copy(dst_hbm.at[pl.ds(0, EC)], dst_b[0], sem_d[0])
    for ci in range(_NCH):
        sbuf = src_b[ci % 2]
        dbuf = dst_b[ci % 2]
        hsn = hdn = None
        if ci + 1 < _NCH:
            nb = (ci + 1) % 2
            off = (ci + 1) * EC
            hsn = pltpu.async_copy(src_hbm.at[pl.ds(off, EC)],
                                   src_b[nb], sem_s[nb])
            hdn = pltpu.async_copy(dst_hbm.at[pl.ds(off, EC)],
                                   dst_b[nb], sem_d[nb])
        hs.wait()
        hd.wait()

        def _body(i, cc, sbuf=sbuf, dbuf=dbuf):
            s16s = [sbuf[pl.ds((i * _U + u) * 16, 16)] for u in range(_U)]
            d16s = [dbuf[pl.ds((i * _U + u) * 16, 16)] for u in range(_U)]
            vs = [plsc.load_gather(g_v, [s16]) for s16 in s16s]
            for d16, v in zip(d16s, vs):
                plsc.addupdate_scatter(acc_v, [d16], v)
            return cc
        lax.fori_loop(0, EC // 16 // _U, _body, 0)
        hs, hd = hsn, hdn

    widv = jnp.full((16,), 0, jnp.int32) + wid
    bgd = plsc.load_gather(bg_v, [widv])

    def _fin(i, c):
        idx = pl.ds(i * 16, 16)
        hv = dinv_v[idx] * (acc_v[idx] + g_v[idx]) + bgd
        acc_v[idx] = jnp.maximum(hv, 0.0)
        return c
    lax.fori_loop(0, N // 16, _fin, 0)

    pltpu.sync_copy(acc_v, out_hbm.at[wid])


def _fc1_body(h_ref, w_ref, o_ref):
    i = pl.program_id(0)

    @pl.when(i == 0)
    def _():
        o_ref[...] = jnp.zeros_like(o_ref)

    o_ref[...] += jnp.dot(h_ref[...], w_ref[...],
                          preferred_element_type=jnp.float32)


def _k3(h2, W1):
    nblk = (N * D_H) // KBLK
    return pl.pallas_call(
        _fc1_body,
        grid=(nblk,),
        in_specs=[
            pl.BlockSpec((1, KBLK), lambda i: (0, i)),
            pl.BlockSpec((KBLK, 128), lambda i: (i, 0)),
        ],
        out_specs=pl.BlockSpec((1, 128), lambda i: (0, 0)),
        out_shape=jax.ShapeDtypeStruct((1, 128), jnp.float32),
    )(h2, W1)


def _fc2_body(h_ref, b1_ref, w2_ref, b2_ref, o_ref):
    h3 = jnp.maximum(h_ref[...] + b1_ref[...], 0.0)
    o_ref[...] = jnp.maximum(
        jnp.dot(h3, w2_ref[...], preferred_element_type=jnp.float32)
        + b2_ref[...], 0.0)


def _k4(h3pre, b1, W2, b2):
    return pl.pallas_call(
        _fc2_body,
        out_shape=jax.ShapeDtypeStruct((1, N), jnp.float32),
    )(h3pre, b1.reshape(1, 128), W2, b2.reshape(1, N))


def kernel(x, edge_index, Wg, bg, W1, b1, W2, b2):
    src = edge_index[0]
    dst = edge_index[1]
    xw = _k1(x, Wg)
    h1T = _sc_gcn(src, dst, xw.T, bg)
    h2 = h1T.T.reshape(1, N * D_H)
    h3pre = _k3(h2, W1)
    return _k4(h3pre, b1, W2, b2)


# speedup vs baseline: 22.0931x; 1.0107x over previous
import functools

import jax
import jax.numpy as jnp
from jax import lax
from jax.experimental import pallas as pl
from jax.experimental.pallas import tpu as pltpu
from jax.experimental.pallas import tpu_sc as plsc

N = 10000
E = 160000
D_IN = 128
D_H = 32
EC = 16000
KBLK = 16000


def _mm1_body(x_ref, wg_ref, o_ref):
    o_ref[...] = jnp.dot(x_ref[...], wg_ref[...],
                         preferred_element_type=jnp.float32)


def _k1(x, Wg):
    return pl.pallas_call(
        _mm1_body,
        out_shape=jax.ShapeDtypeStruct((N, D_H), jnp.float32),
    )(x, Wg)


_MESH = plsc.VectorSubcoreMesh(core_axis_name="c", subcore_axis_name="s")

_NCH = E // EC
_U = 8
assert E % EC == 0 and EC % (16 * _U) == 0 and N % 16 == 0


@functools.partial(
    pl.kernel,
    mesh=_MESH,
    compiler_params=pltpu.CompilerParams(needs_layout_passes=False),
    out_type=jax.ShapeDtypeStruct((D_H, N), jnp.float32),
    scratch_types=[
        pltpu.VMEM((EC,), jnp.int32),
        pltpu.VMEM((EC,), jnp.int32),
        pltpu.VMEM((EC,), jnp.int32),
        pltpu.VMEM((EC,), jnp.int32),
        pltpu.VMEM((N,), jnp.float32),
        pltpu.VMEM((N,), jnp.float32),
        pltpu.VMEM((N,), jnp.float32),
        pltpu.VMEM((D_H,), jnp.float32),
        pltpu.SemaphoreType.DMA,
        pltpu.SemaphoreType.DMA,
        pltpu.SemaphoreType.DMA,
        pltpu.SemaphoreType.DMA,
    ],
)
def _sc_gcn(src_hbm, dst_hbm, xwT_hbm, bg_hbm, out_hbm,
            src_v0, src_v1, dst_v0, dst_v1, dinv_v, g_v, acc_v, bg_v,
            sem_s0, sem_s1, sem_d0, sem_d1):
    wid = lax.axis_index("s") * 2 + lax.axis_index("c")
    src_b = (src_v0, src_v1)
    dst_b = (dst_v0, dst_v1)
    sem_s = (sem_s0, sem_s1)
    sem_d = (sem_d0, sem_d1)

    pltpu.sync_copy(xwT_hbm.at[wid], g_v)
    pltpu.sync_copy(bg_hbm, bg_v)

    one16 = jnp.full((16,), 1.0, jnp.float32)
    zero16 = jnp.zeros((16,), jnp.float32)

    def _init(i, c):
        idx = pl.ds(i * 16, 16)
        dinv_v[idx] = one16
        acc_v[idx] = zero16
        return c
    lax.fori_loop(0, N // 16, _init, 0)

    h = pltpu.async_copy(dst_hbm.at[pl.ds(0, EC)], dst_b[0], sem_d[0])
    for ci in range(_NCH):
        buf = dst_b[ci % 2]
        hn = None
        if ci + 1 < _NCH:
            nb = (ci + 1) % 2
            hn = pltpu.async_copy(dst_hbm.at[pl.ds((ci + 1) * EC, EC)],
                                  dst_b[nb], sem_d[nb])
        h.wait()

        def _body(i, cc, buf=buf):
            d16s = [buf[pl.ds((i * _U + u) * 16, 16)] for u in range(_U)]
            for d16 in d16s:
                plsc.addupdate_scatter(dinv_v, [d16], one16)
            return cc
        lax.fori_loop(0, EC // 16 // _U, _body, 0)
        h = hn

    def _rsqrt(i, c):
        idx = pl.ds(i * 16, 16)
        v = dinv_v[idx]
        yi = jnp.int32(0x5F3759DF) - (plsc.bitcast(v, jnp.int32) >> 1)
        y = plsc.bitcast(yi, jnp.float32)
        hf = v * 0.5
        y = y * (1.5 - hf * y * y)
        y = y * (1.5 - hf * y * y)
        y = y * (1.5 - hf * y * y)
        dinv_v[idx] = y
        g_v[idx] = y * g_v[idx]
        return c
    lax.fori_loop(0, N // 16, _rsqrt, 0)

    hs = pltpu.async_copy(src_hbm.at[pl.ds(0, EC)], src_b[0], sem_s[0])
    hd = pltpu.async_copy(dst_hbm.at[pl.ds(0, EC)], dst_b[0], sem_d[0])
    for ci in range(_NCH):
        sbuf = src_b[ci % 2]
        dbuf = dst_b[ci % 2]
        hsn = hdn = None
        if ci + 1 < _NCH:
            nb = (ci + 1) % 2
            off = (ci + 1) * EC
            hsn = pltpu.async_copy(src_hbm.at[pl.ds(off, EC)],
                                   src_b[nb], sem_s[nb])
            hdn = pltpu.async_copy(dst_hbm.at[pl.ds(off, EC)],
                                   dst_b[nb], sem_d[nb])
        hs.wait()
        hd.wait()

        def _body(i, cc, sbuf=sbuf, dbuf=dbuf):
            s16s = [sbuf[pl.ds((i * _U + u) * 16, 16)] for u in range(_U)]
            d16s = [dbuf[pl.ds((i * _U + u) * 16, 16)] for u in range(_U)]
            vs = [plsc.load_gather(g_v, [s16]) for s16 in s16s]
            for d16, v in zip(d16s, vs):
                plsc.addupdate_scatter(acc_v, [d16], v)
            return cc
        lax.fori_loop(0, EC // 16 // _U, _body, 0)
        hs, hd = hsn, hdn

    widv = jnp.full((16,), 0, jnp.int32) + wid
    bgd = plsc.load_gather(bg_v, [widv])

    def _fin(i, c):
        idx = pl.ds(i * 16, 16)
        hv = dinv_v[idx] * (acc_v[idx] + g_v[idx]) + bgd
        acc_v[idx] = jnp.maximum(hv, 0.0)
        return c
    lax.fori_loop(0, N // 16, _fin, 0)

    pltpu.sync_copy(acc_v, out_hbm.at[wid])


def _fc1_body(h_ref, w_ref, o_ref):
    i = pl.program_id(0)

    @pl.when(i == 0)
    def _():
        o_ref[...] = jnp.zeros_like(o_ref)

    o_ref[...] += jnp.dot(h_ref[...], w_ref[...],
                          preferred_element_type=jnp.float32)


def _k3(h2, W1):
    nblk = (N * D_H) // KBLK
    return pl.pallas_call(
        _fc1_body,
        grid=(nblk,),
        in_specs=[
            pl.BlockSpec((1, KBLK), lambda i: (0, i)),
            pl.BlockSpec((KBLK, 128), lambda i: (i, 0)),
        ],
        out_specs=pl.BlockSpec((1, 128), lambda i: (0, 0)),
        out_shape=jax.ShapeDtypeStruct((1, 128), jnp.float32),
    )(h2, W1)


def _fc2_body(h_ref, b1_ref, w2_ref, b2_ref, o_ref):
    h3 = jnp.maximum(h_ref[...] + b1_ref[...], 0.0)
    o_ref[...] = jnp.maximum(
        jnp.dot(h3, w2_ref[...], preferred_element_type=jnp.float32)
        + b2_ref[...], 0.0)


def _k4(h3pre, b1, W2, b2):
    return pl.pallas_call(
        _fc2_body,
        out_shape=jax.ShapeDtypeStruct((1, N), jnp.float32),
    )(h3pre, b1.reshape(1, 128), W2, b2.reshape(1, N))


def kernel(x, edge_index, Wg, bg, W1, b1, W2, b2):
    src = edge_index[0]
    dst = edge_index[1]
    xw = _k1(x, Wg)
    h1T = _sc_gcn(src, dst, xw.T, bg)
    h2 = h1T.T.reshape(1, N * D_H)
    h3pre = _k3(h2, W1)
    return _k4(h3pre, b1, W2, b2)


# speedup vs baseline: 23.6493x; 1.0704x over previous
import functools

import jax
import jax.numpy as jnp
from jax import lax
from jax.experimental import pallas as pl
from jax.experimental.pallas import tpu as pltpu
from jax.experimental.pallas import tpu_sc as plsc

N = 10000
E = 160000
D_IN = 128
D_H = 32
EC = 16000
KBLK = 16000


_SHIFT = 14


def _mm1_body(x_ref, wg_ref, e_ref, o_ref, p_ref):
    o_ref[...] = lax.dot_general(
        wg_ref[...], x_ref[...],
        dimension_numbers=(((0,), (1,)), ((), ())),
        preferred_element_type=jnp.float32)
    p_ref[...] = (e_ref[0:1, :] << _SHIFT) | e_ref[1:2, :]


def _k1(x, Wg, edge_index):
    return pl.pallas_call(
        _mm1_body,
        out_shape=(
            jax.ShapeDtypeStruct((D_H, N), jnp.float32),
            jax.ShapeDtypeStruct((1, E), jnp.int32),
        ),
    )(x, Wg, edge_index)


_MESH = plsc.VectorSubcoreMesh(core_axis_name="c", subcore_axis_name="s")

_NCH = E // EC
_U = 8
assert E % EC == 0 and EC % (16 * _U) == 0 and N % 16 == 0


@functools.partial(
    pl.kernel,
    mesh=_MESH,
    compiler_params=pltpu.CompilerParams(needs_layout_passes=False),
    out_type=jax.ShapeDtypeStruct((D_H, N), jnp.float32),
    scratch_types=[
        pltpu.VMEM((EC,), jnp.int32),
        pltpu.VMEM((EC,), jnp.int32),
        pltpu.VMEM((N,), jnp.float32),
        pltpu.VMEM((N,), jnp.float32),
        pltpu.VMEM((N,), jnp.float32),
        pltpu.VMEM((D_H,), jnp.float32),
        pltpu.SemaphoreType.DMA,
        pltpu.SemaphoreType.DMA,
    ],
)
def _sc_gcn(ep_hbm, xwT_hbm, bg_hbm, out_hbm,
            ep_v0, ep_v1, dinv_v, g_v, acc_v, bg_v,
            sem_0, sem_1):
    wid = lax.axis_index("s") * 2 + lax.axis_index("c")
    ep_b = (ep_v0, ep_v1)
    sem = (sem_0, sem_1)
    dmask = jnp.full((16,), (1 << _SHIFT) - 1, jnp.int32)

    pltpu.sync_copy(xwT_hbm.at[wid], g_v)
    pltpu.sync_copy(bg_hbm, bg_v)

    one16 = jnp.full((16,), 1.0, jnp.float32)
    zero16 = jnp.zeros((16,), jnp.float32)

    def _init(i, c):
        idx = pl.ds(i * 16, 16)
        dinv_v[idx] = one16
        acc_v[idx] = zero16
        return c
    lax.fori_loop(0, N // 16, _init, 0)

    h = pltpu.async_copy(ep_hbm.at[pl.ds(0, EC)], ep_b[0], sem[0])
    for ci in range(_NCH):
        buf = ep_b[ci % 2]
        hn = None
        if ci + 1 < _NCH:
            nb = (ci + 1) % 2
            hn = pltpu.async_copy(ep_hbm.at[pl.ds((ci + 1) * EC, EC)],
                                  ep_b[nb], sem[nb])
        h.wait()

        def _body(i, cc, buf=buf):
            e16s = [buf[pl.ds((i * _U + u) * 16, 16)] for u in range(_U)]
            for e16 in e16s:
                plsc.addupdate_scatter(dinv_v, [e16 & dmask], one16)
            return cc
        lax.fori_loop(0, EC // 16 // _U, _body, 0)
        h = hn

    def _rsqrt(i, c):
        idx = pl.ds(i * 16, 16)
        v = dinv_v[idx]
        yi = jnp.int32(0x5F3759DF) - (plsc.bitcast(v, jnp.int32) >> 1)
        y = plsc.bitcast(yi, jnp.float32)
        hf = v * 0.5
        y = y * (1.5 - hf * y * y)
        y = y * (1.5 - hf * y * y)
        y = y * (1.5 - hf * y * y)
        dinv_v[idx] = y
        g_v[idx] = y * g_v[idx]
        return c
    lax.fori_loop(0, N // 16, _rsqrt, 0)

    h = pltpu.async_copy(ep_hbm.at[pl.ds(0, EC)], ep_b[0], sem[0])
    for ci in range(_NCH):
        buf = ep_b[ci % 2]
        hn = None
        if ci + 1 < _NCH:
            nb = (ci + 1) % 2
            hn = pltpu.async_copy(ep_hbm.at[pl.ds((ci + 1) * EC, EC)],
                                  ep_b[nb], sem[nb])
        h.wait()

        def _body(i, cc, buf=buf):
            e16s = [buf[pl.ds((i * _U + u) * 16, 16)] for u in range(_U)]
            vs = [plsc.load_gather(g_v, [e16 >> _SHIFT]) for e16 in e16s]
            for e16, v in zip(e16s, vs):
                plsc.addupdate_scatter(acc_v, [e16 & dmask], v)
            return cc
        lax.fori_loop(0, EC // 16 // _U, _body, 0)
        h = hn

    widv = jnp.full((16,), 0, jnp.int32) + wid
    bgd = plsc.load_gather(bg_v, [widv])

    def _fin(i, c):
        idx = pl.ds(i * 16, 16)
        hv = dinv_v[idx] * (acc_v[idx] + g_v[idx]) + bgd
        acc_v[idx] = jnp.maximum(hv, 0.0)
        return c
    lax.fori_loop(0, N // 16, _fin, 0)

    pltpu.sync_copy(acc_v, out_hbm.at[wid])


def _fc1_body(h_ref, w_ref, o_ref):
    i = pl.program_id(0)

    @pl.when(i == 0)
    def _():
        o_ref[...] = jnp.zeros_like(o_ref)

    o_ref[...] += jnp.dot(h_ref[...], w_ref[...],
                          preferred_element_type=jnp.float32)


def _k3(h2, W1):
    nblk = (N * D_H) // KBLK
    return pl.pallas_call(
        _fc1_body,
        grid=(nblk,),
        in_specs=[
            pl.BlockSpec((1, KBLK), lambda i: (0, i)),
            pl.BlockSpec((KBLK, 128), lambda i: (i, 0)),
        ],
        out_specs=pl.BlockSpec((1, 128), lambda i: (0, 0)),
        out_shape=jax.ShapeDtypeStruct((1, 128), jnp.float32),
    )(h2, W1)


def _fc2_body(h_ref, b1_ref, w2_ref, b2_ref, o_ref):
    h3 = jnp.maximum(h_ref[...] + b1_ref[...], 0.0)
    o_ref[...] = jnp.maximum(
        jnp.dot(h3, w2_ref[...], preferred_element_type=jnp.float32)
        + b2_ref[...], 0.0)


def _k4(h3pre, b1, W2, b2):
    return pl.pallas_call(
        _fc2_body,
        out_shape=jax.ShapeDtypeStruct((1, N), jnp.float32),
    )(h3pre, b1.reshape(1, 128), W2, b2.reshape(1, N))


def kernel(x, edge_index, Wg, bg, W1, b1, W2, b2):
    xwT, epacked = _k1(x, Wg, edge_index)
    h1T = _sc_gcn(epacked.reshape(E), xwT, bg)
    h2 = h1T.T.reshape(1, N * D_H)
    h3pre = _k3(h2, W1)
    return _k4(h3pre, b1, W2, b2)


# speedup vs baseline: 28.6203x; 1.2102x over previous
import functools

import jax
import jax.numpy as jnp
from jax import lax
from jax.experimental import pallas as pl
from jax.experimental.pallas import tpu as pltpu
from jax.experimental.pallas import tpu_sc as plsc

N = 10000
E = 160000
D_IN = 128
D_H = 32
EC = 16000
KBLK = 16000
EPT = E // 32

_SHIFT = 14
_NCH = E // EC
_U = 8
assert E % EC == 0 and EC % (16 * _U) == 0 and N % 16 == 0
assert E % 32 == 0 and EPT % 8 == 0

_MESH = plsc.VectorSubcoreMesh(core_axis_name="c", subcore_axis_name="s")
_SC_PARAMS = pltpu.CompilerParams(needs_layout_passes=False)


def _k1_body(x_ref, wg_ref, e_ref, o_ref, p_ref):
    o_ref[...] = lax.dot_general(
        wg_ref[...], x_ref[...],
        dimension_numbers=(((0,), (1,)), ((), ())),
        preferred_element_type=jnp.float32)
    p_ref[...] = (e_ref[0:1, :] << _SHIFT) | e_ref[1:2, :]


def _k1(x, Wg, edge_index):
    return pl.pallas_call(
        _k1_body,
        out_shape=(
            jax.ShapeDtypeStruct((D_H, N), jnp.float32),
            jax.ShapeDtypeStruct((1, E), jnp.int32),
        ),
    )(x, Wg, edge_index)


@functools.partial(
    pl.kernel,
    mesh=_MESH,
    compiler_params=_SC_PARAMS,
    out_type=jax.ShapeDtypeStruct((32, N), jnp.float32),
    scratch_types=[
        pltpu.VMEM((EPT,), jnp.int32),
        pltpu.VMEM((N,), jnp.float32),
        pltpu.SemaphoreType.DMA,
    ],
)
def _sc_deg(ep_hbm, out_hbm, ep_v, deg_v, sem_e):
    wid = lax.axis_index("s") * 2 + lax.axis_index("c")

    h = pltpu.async_copy(ep_hbm.at[pl.ds(wid * EPT, EPT)], ep_v, sem_e)

    zero16 = jnp.zeros((16,), jnp.float32)

    def _init(i, c):
        deg_v[pl.ds(i * 16, 16)] = zero16
        return c
    lax.fori_loop(0, N // 16, _init, 0)
    h.wait()

    one16 = jnp.full((16,), 1.0, jnp.float32)
    dmask = jnp.full((16,), (1 << _SHIFT) - 1, jnp.int32)
    nfull = EPT // 16
    rem = EPT - nfull * 16

    def _body(i, c):
        e16 = ep_v[pl.ds(i * 16, 16)]
        plsc.addupdate_scatter(deg_v, [e16 & dmask], one16)
        return c
    lax.fori_loop(0, nfull, _body, 0)

    if rem:
        e16 = ep_v[pl.ds(EPT - 16, 16)]
        mask = lax.iota(jnp.int32, 16) >= (16 - rem)
        plsc.addupdate_scatter(deg_v, [e16 & dmask], one16, mask=mask)

    pltpu.sync_copy(deg_v, out_hbm.at[wid])


def _kd_body(degp_ref, xwt_ref, dinv_ref, gall_ref):
    deg = 1.0 + jnp.sum(degp_ref[...], axis=0, keepdims=True)
    dinv = lax.rsqrt(deg)
    dinv_ref[...] = dinv
    gall_ref[...] = dinv * xwt_ref[...]


def _kd(degP, xwT):
    return pl.pallas_call(
        _kd_body,
        out_shape=(
            jax.ShapeDtypeStruct((1, N), jnp.float32),
            jax.ShapeDtypeStruct((D_H, N), jnp.float32),
        ),
    )(degP, xwT)


@functools.partial(
    pl.kernel,
    mesh=_MESH,
    compiler_params=_SC_PARAMS,
    out_type=jax.ShapeDtypeStruct((D_H, N), jnp.float32),
    scratch_types=[
        pltpu.VMEM((EC,), jnp.int32),
        pltpu.VMEM((EC,), jnp.int32),
        pltpu.VMEM((N,), jnp.float32),
        pltpu.VMEM((N,), jnp.float32),
        pltpu.SemaphoreType.DMA,
        pltpu.SemaphoreType.DMA,
        pltpu.SemaphoreType.DMA,
    ],
)
def _sc_msg(ep_hbm, gall_hbm, out_hbm, ep_v0, ep_v1, g_v, acc_v,
            sem_0, sem_1, sem_g):
    wid = lax.axis_index("s") * 2 + lax.axis_index("c")
    ep_b = (ep_v0, ep_v1)
    sem = (sem_0, sem_1)
    dmask = jnp.full((16,), (1 << _SHIFT) - 1, jnp.int32)

    hg = pltpu.async_copy(gall_hbm.at[wid], g_v, sem_g)
    h = pltpu.async_copy(ep_hbm.at[pl.ds(0, EC)], ep_b[0], sem[0])

    zero16 = jnp.zeros((16,), jnp.float32)

    def _init(i, c):
        acc_v[pl.ds(i * 16, 16)] = zero16
        return c
    lax.fori_loop(0, N // 16, _init, 0)
    hg.wait()

    for ci in range(_NCH):
        buf = ep_b[ci % 2]
        hn = None
        if ci + 1 < _NCH:
            nb = (ci + 1) % 2
            hn = pltpu.async_copy(ep_hbm.at[pl.ds((ci + 1) * EC, EC)],
                                  ep_b[nb], sem[nb])
        h.wait()

        def _body(i, cc, buf=buf):
            e16s = [buf[pl.ds((i * _U + u) * 16, 16)] for u in range(_U)]
            vs = [plsc.load_gather(g_v, [e16 >> _SHIFT]) for e16 in e16s]
            for e16, v in zip(e16s, vs):
                plsc.addupdate_scatter(acc_v, [e16 & dmask], v)
            return cc
        lax.fori_loop(0, EC // 16 // _U, _body, 0)
        h = hn

    pltpu.sync_copy(acc_v, out_hbm.at[wid])


def _k2c_body(acc_ref, gall_ref, dinv_ref, bg_ref, o_ref):
    o_ref[...] = jnp.maximum(
        dinv_ref[...] * (acc_ref[...] + gall_ref[...]) + bg_ref[...], 0.0)


def _k2c(accT, gall, dinv, bg):
    return pl.pallas_call(
        _k2c_body,
        out_shape=jax.ShapeDtypeStruct((D_H, N), jnp.float32),
    )(accT, gall, dinv, bg.reshape(D_H, 1))


def _fc1_body(h_ref, w_ref, o_ref):
    i = pl.program_id(0)

    @pl.when(i == 0)
    def _():
        o_ref[...] = jnp.zeros_like(o_ref)

    o_ref[...] += jnp.dot(h_ref[...], w_ref[...],
                          preferred_element_type=jnp.float32)


def _k3(h2, W1):
    nblk = (N * D_H) // KBLK
    return pl.pallas_call(
        _fc1_body,
        grid=(nblk,),
        in_specs=[
            pl.BlockSpec((1, KBLK), lambda i: (0, i)),
            pl.BlockSpec((KBLK, 128), lambda i: (i, 0)),
        ],
        out_specs=pl.BlockSpec((1, 128), lambda i: (0, 0)),
        out_shape=jax.ShapeDtypeStruct((1, 128), jnp.float32),
    )(h2, W1)


def _fc2_body(h_ref, b1_ref, w2_ref, b2_ref, o_ref):
    h3 = jnp.maximum(h_ref[...] + b1_ref[...], 0.0)
    o_ref[...] = jnp.maximum(
        jnp.dot(h3, w2_ref[...], preferred_element_type=jnp.float32)
        + b2_ref[...], 0.0)


def _k4(h3pre, b1, W2, b2):
    return pl.pallas_call(
        _fc2_body,
        out_shape=jax.ShapeDtypeStruct((1, N), jnp.float32),
    )(h3pre, b1.reshape(1, 128), W2, b2.reshape(1, N))


def kernel(x, edge_index, Wg, bg, W1, b1, W2, b2):
    xwT, epacked = _k1(x, Wg, edge_index)
    ep = epacked.reshape(E)
    degP = _sc_deg(ep)
    dinv, gall = _kd(degP, xwT)
    accT = _sc_msg(ep, gall)
    h1T = _k2c(accT, gall, dinv, bg)
    h2 = h1T.T.reshape(1, N * D_H)
    h3pre = _k3(h2, W1)
    return _k4(h3pre, b1, W2, b2)
